# Initial kernel scaffold; baseline (speedup 1.0000x reference)
import jax
import jax.numpy as jnp
from jax.experimental import pallas as pl


def kernel(node_feat, edge_feat, edge_index, node_val_idx, g_idx, w_n2l_W, w_n2l_b, node_val_emb, we_W, we_b, conv_W, conv_b, l2_W, l2_b, msg_g, msg_b, hid_g, hid_b, ro_W, ro_b):
    raise NotImplementedError("write your pallas kernel here")


# speedup vs baseline: 7.8766x; 7.8766x over previous
import functools

import jax
import jax.numpy as jnp
from jax import lax
from jax.experimental import pallas as pl
from jax.experimental.pallas import tpu as pltpu
from jax.experimental.pallas import tpu_sc as plsc

N = 10000
E = 320000
DF = 128
DE = 16
LAT = 128
ODIM = 64
NVAL = 1000
MAXLV = 3
NG = 16

NC = 2
NS = 16
NW = NC * NS
CH = 128
CPW = 80
EP = NW * CPW * CH
PADR = 368
NP = N + PADR
RPT = NP // NS
NO = 10240
OPT = NO // NS
NVC = 3
NVP = NW * NVC * CH
EPS = 1e-5

_mesh = plsc.VectorSubcoreMesh(core_axis_name="c", subcore_axis_name="s")
_f32 = jnp.float32


def _zero_fill(buf, nrow, ncol):
    z16 = jnp.zeros((16,), _f32)

    def body(i, _):
        for j in range(ncol // 16):
            buf[i, pl.ds(j * 16, 16)] = z16
        return 0

    lax.fori_loop(0, nrow, body, 0)


def _zero_spmem_stripe(acc_sh, zsrc, sid, width):
    base = sid * RPT
    for k in range(RPT // CH):
        pltpu.sync_copy(zsrc.at[pl.ds(0, CH)], acc_sh.at[pl.ds(base + k * CH, CH)])
    rem = RPT % CH
    if rem:
        pltpu.sync_copy(zsrc.at[pl.ds(0, rem)],
                        acc_sh.at[pl.ds(base + (RPT // CH) * CH, rem)])


def _agg_body(h_hbm, src_hbm, dst_hbm, out_hbm, src_v, dst_v, rows_v, acc_sh,
              gsem):
    cid = lax.axis_index("c")
    sid = lax.axis_index("s")
    wid = sid * NC + cid

    _zero_fill(rows_v, CH, LAT)
    _zero_spmem_stripe(acc_sh, rows_v, sid, LAT)
    plsc.subcore_barrier()

    pltpu.sync_copy(src_hbm.at[wid], src_v)
    pltpu.sync_copy(dst_hbm.at[wid], dst_v)

    def chunk(j, _):
        pltpu.async_copy(h_hbm.at[src_v.at[j]], rows_v, gsem).wait()
        pltpu.sync_copy(rows_v, acc_sh.at[dst_v.at[j]], add=True)
        return 0

    lax.fori_loop(0, CPW, chunk, 0)

    plsc.subcore_barrier()
    pltpu.sync_copy(acc_sh.at[pl.ds(sid * OPT, OPT)],
                    out_hbm.at[cid, pl.ds(sid * OPT, OPT)])


_agg_call = functools.partial(
    pl.kernel,
    out_type=jax.ShapeDtypeStruct((NC, NO, LAT), _f32),
    mesh=_mesh,
    scratch_types=[
        pltpu.VMEM((CPW, CH), jnp.int32),
        pltpu.VMEM((CPW, CH), jnp.int32),
        pltpu.VMEM((CH, LAT), _f32),
        pltpu.VMEM_SHARED((NP, LAT), _f32),
        pltpu.SemaphoreType.DMA,
    ],
)(_agg_body)


def _pro_body(ef_hbm, dst_hbm, nv_hbm, tab_hbm, pool_out, emb_out,
              dst_v, ef_v, nv_v, emb_v, pool_sh, gsem):
    cid = lax.axis_index("c")
    sid = lax.axis_index("s")
    wid = sid * NC + cid

    _zero_fill(ef_v, CH, LAT)
    _zero_spmem_stripe(pool_sh, ef_v, sid, LAT)
    plsc.subcore_barrier()

    pltpu.sync_copy(dst_hbm.at[wid], dst_v)

    def chunk(j, _):
        pltpu.sync_copy(ef_hbm.at[wid, j], ef_v)
        pltpu.sync_copy(ef_v, pool_sh.at[dst_v.at[j]], add=True)
        return 0

    lax.fori_loop(0, CPW, chunk, 0)

    pltpu.sync_copy(nv_hbm.at[wid], nv_v)
    for j in range(NVC):
        pltpu.async_copy(tab_hbm.at[nv_v.at[j]], emb_v, gsem).wait()
        pltpu.sync_copy(emb_v, emb_out.at[pl.ds(wid * (NVC * CH) + j * CH, CH)])

    plsc.subcore_barrier()
    pltpu.sync_copy(pool_sh.at[pl.ds(sid * OPT, OPT)],
                    pool_out.at[cid, pl.ds(sid * OPT, OPT)])


_pro_call = functools.partial(
    pl.kernel,
    out_type=(jax.ShapeDtypeStruct((NC, NO, LAT), _f32),
              jax.ShapeDtypeStruct((NVP, LAT), _f32)),
    mesh=_mesh,
    scratch_types=[
        pltpu.VMEM((CPW, CH), jnp.int32),
        pltpu.VMEM((CH, LAT), _f32),
        pltpu.VMEM((NVC, CH), jnp.int32),
        pltpu.VMEM((CH, LAT), _f32),
        pltpu.VMEM_SHARED((NP, LAT), _f32),
        pltpu.SemaphoreType.DMA,
    ],
)(_pro_body)


def _bn_tc(x, g, b):
    m = jnp.mean(x, axis=0, keepdims=True)
    v = jnp.mean((x - m) ** 2, axis=0, keepdims=True)
    return (x - m) / jnp.sqrt(v + EPS) * g + b


def _h0_body(nf, emb, pp, wn, wnb, w0e, g0, b0, h_out, pool_out):
    pool = pp[0, 0:N, :] + pp[1, 0:N, :]
    x = jnp.dot(nf[...], wn[...], preferred_element_type=_f32)
    x = x + wnb[...] + emb[...] + jnp.dot(pool, w0e[...],
                                          preferred_element_type=_f32)
    x = jnp.maximum(x, 0.0)
    h_out[...] = _bn_tc(x, g0[...], b0[...])
    pool_out[...] = pool


def _upd_core(h_ref, a_ref, pool, cw, cb, wle, l2w, l2b, hg, hb, mg, mb):
    agg = a_ref[0, 0:N, :] + a_ref[1, 0:N, :]
    nl = jnp.dot(agg, cw[...], preferred_element_type=_f32) + cb[...]
    ep = jnp.dot(pool[...], wle[...], preferred_element_type=_f32)
    merged = jnp.maximum(nl + ep, 0.0)
    merged = _bn_tc(merged, hg[...], hb[...])
    x = jnp.dot(merged, l2w[...], preferred_element_type=_f32) + l2b[...] + h_ref[...]
    x = jnp.maximum(x, 0.0)
    return _bn_tc(x, mg[...], mb[...])


def _upd_body(h_ref, a_ref, pool, cw, cb, wle, l2w, l2b, hg, hb, mg, mb, out):
    out[...] = _upd_core(h_ref, a_ref, pool, cw, cb, wle, l2w, l2b, hg, hb,
                         mg, mb)


def _upd_ro_body(h_ref, a_ref, pool, cw, cb, wle, l2w, l2b, hg, hb, mg, mb,
                 gidx, row, rob, out):
    x = _upd_core(h_ref, a_ref, pool, cw, cb, wle, l2w, l2b, hg, hb, mg, mb)
    g = gidx[...]
    rows = []
    for gg in range(NG):
        m = g == gg
        rows.append(jnp.max(jnp.where(m, x, -jnp.inf), axis=0, keepdims=True))
    pooled = jnp.concatenate(rows, axis=0)
    out[...] = jnp.maximum(
        jnp.dot(pooled, row[...], preferred_element_type=_f32) + rob[...], 0.0)


def kernel(node_feat, edge_feat, edge_index, node_val_idx, g_idx,
           w_n2l_W, w_n2l_b, node_val_emb, we_W, we_b,
           conv_W, conv_b, l2_W, l2_b,
           msg_g, msg_b, hid_g, hid_b, ro_W, ro_b):
    src = edge_index[0].astype(jnp.int32)
    dst = edge_index[1].astype(jnp.int32)
    pad = EP - E
    ar = jnp.arange(pad, dtype=jnp.int32)
    src3 = jnp.concatenate([src, (ar * 131) % N]).reshape(NW, CPW, CH)
    dst3 = jnp.concatenate([dst, N + (ar % PADR)]).reshape(NW, CPW, CH)

    ef2 = jnp.zeros((EP, LAT), _f32)
    ef2 = ef2.at[:E, :DE].set(edge_feat).at[:E, DE].set(1.0)
    ef4 = ef2.reshape(NW, CPW, CH, LAT)

    nvp = jnp.concatenate([
        node_val_idx.astype(jnp.int32),
        (jnp.arange(NVP - N, dtype=jnp.int32) * 7) % NVAL,
    ]).reshape(NW, NVC, CH)

    pool_p, emb_p = _pro_call(ef4, dst3, nvp, node_val_emb)
    emb = emb_p[:N]

    wext = jnp.zeros((MAXLV + 1, LAT, LAT), _f32)
    wext = wext.at[:, :DE, :].set(we_W).at[:, DE, :].set(we_b)

    r = lambda a: a.reshape(1, -1)

    h0_call = pl.pallas_call(
        _h0_body,
        out_shape=(jax.ShapeDtypeStruct((N, LAT), _f32),
                   jax.ShapeDtypeStruct((N, LAT), _f32)),
    )
    h, pool_sum = h0_call(node_feat, emb, pool_p, w_n2l_W, r(w_n2l_b),
                          wext[0], r(msg_g[0]), r(msg_b[0]))

    out = None
    for lv in range(MAXLV):
        a = _agg_call(h, src3, dst3)
        args = (h, a, pool_sum, conv_W[lv], r(conv_b[lv]), wext[lv + 1],
                l2_W[lv], r(l2_b[lv]), r(hid_g[lv]), r(hid_b[lv]),
                r(msg_g[lv + 1]), r(msg_b[lv + 1]))
        if lv < MAXLV - 1:
            h = pl.pallas_call(
                _upd_body,
                out_shape=jax.ShapeDtypeStruct((N, LAT), _f32),
            )(*args)
        else:
            out = pl.pallas_call(
                _upd_ro_body,
                out_shape=jax.ShapeDtypeStruct((NG, ODIM), _f32),
            )(*args, g_idx.reshape(N, 1).astype(jnp.int32), ro_W, r(ro_b))
    return out


# speedup vs baseline: 9.3982x; 1.1932x over previous
import functools

import jax
import jax.numpy as jnp
from jax import lax
from jax.experimental import pallas as pl
from jax.experimental.pallas import tpu as pltpu
from jax.experimental.pallas import tpu_sc as plsc

N = 10000
E = 320000
DF = 128
DE = 16
LAT = 128
ODIM = 64
NVAL = 1000
MAXLV = 3
NG = 16

NC = 2
NS = 16
NW = NC * NS
CH = 128
CPW = 80
HB = 40
EP = NW * CPW * CH
PADR = 368
NP = N + PADR
RPT = NP // NS
NO = 10240
OPT = NO // NS
NVC = 3
NVP = NW * NVC * CH
EPS = 1e-5

_mesh = plsc.VectorSubcoreMesh(core_axis_name="c", subcore_axis_name="s")
_f32 = jnp.float32


def _zero_fill(buf, nrow, ncol):
    z16 = jnp.zeros((16,), _f32)

    def body(i, _):
        for j in range(ncol // 16):
            buf[i, pl.ds(j * 16, 16)] = z16
        return 0

    lax.fori_loop(0, nrow, body, 0)


def _zero_spmem_stripe(acc_sh, zsrc, sid, width):
    base = sid * RPT
    for k in range(RPT // CH):
        pltpu.sync_copy(zsrc.at[pl.ds(0, CH)], acc_sh.at[pl.ds(base + k * CH, CH)])
    rem = RPT % CH
    if rem:
        pltpu.sync_copy(zsrc.at[pl.ds(0, rem)],
                        acc_sh.at[pl.ds(base + (RPT // CH) * CH, rem)])


def _agg_body(h_hbm, src_hbm, dst_hbm, out_hbm, src_v, dst_v, rows_v, acc_sh,
              gsem, gsem2):
    cid = lax.axis_index("c")
    sid = lax.axis_index("s")
    wid = sid * NC + cid

    _zero_fill(rows_v.at[0], CH, LAT)
    _zero_spmem_stripe(acc_sh, rows_v.at[0], sid, LAT)
    plsc.subcore_barrier()

    def g_issue(j, b, sem):
        return pltpu.async_copy(h_hbm.at[src_v.at[j]], rows_v.at[b], sem)

    def g_wait(j, b, sem):
        pltpu.make_async_copy(h_hbm.at[src_v.at[j]], rows_v.at[b], sem).wait()

    for s in range(CPW // HB):
        pltpu.sync_copy(src_hbm.at[wid, pl.ds(s * HB, HB)], src_v)
        pltpu.sync_copy(dst_hbm.at[wid, pl.ds(s * HB, HB)], dst_v)
        g_issue(0, 0, gsem)

        def pair(i, _):
            j0 = 2 * i
            j1 = j0 + 1
            g_wait(j0, 0, gsem)
            g_issue(j1, 1, gsem2)
            pltpu.sync_copy(rows_v.at[0], acc_sh.at[dst_v.at[j0]], add=True)
            g_wait(j1, 1, gsem2)

            @pl.when(i < HB // 2 - 1)
            def _():
                g_issue(j0 + 2, 0, gsem)

            pltpu.sync_copy(rows_v.at[1], acc_sh.at[dst_v.at[j1]], add=True)
            return 0

        lax.fori_loop(0, HB // 2, pair, 0)

    plsc.subcore_barrier()
    pltpu.sync_copy(acc_sh.at[pl.ds(sid * OPT, OPT)],
                    out_hbm.at[cid, pl.ds(sid * OPT, OPT)])


_agg_call = functools.partial(
    pl.kernel,
    out_type=jax.ShapeDtypeStruct((NC, NO, LAT), _f32),
    mesh=_mesh,
    scratch_types=[
        pltpu.VMEM((HB, CH), jnp.int32),
        pltpu.VMEM((HB, CH), jnp.int32),
        pltpu.VMEM((2, CH, LAT), _f32),
        pltpu.VMEM_SHARED((NP, LAT), _f32),
        pltpu.SemaphoreType.DMA,
        pltpu.SemaphoreType.DMA,
    ],
)(_agg_body)


def _pro_body(ef_hbm, dst_hbm, nv_hbm, tab_hbm, pool_out, emb_out,
              dst_v, ef_v, efp_v, nv_v, emb_v, pool_sh, gsem):
    cid = lax.axis_index("c")
    sid = lax.axis_index("s")
    wid = sid * NC + cid

    _zero_fill(ef_v, CH, LAT)
    _zero_spmem_stripe(pool_sh, ef_v, sid, LAT)
    plsc.subcore_barrier()

    pltpu.sync_copy(dst_hbm.at[wid], dst_v)

    onehot = jnp.where(lax.iota(jnp.int32, 16) == 0,
                       jnp.float32(1.0), jnp.float32(0.0))
    for i in range(CH):
        ef_v[i, pl.ds(16, 16)] = onehot

    def chunk(j, _):
        pltpu.sync_copy(ef_hbm.at[wid, j], efp_v)
        for i in range(CH):
            ef_v[i, pl.ds(0, 16)] = efp_v[i // 8, pl.ds((i % 8) * 16, 16)]
        pltpu.sync_copy(ef_v, pool_sh.at[dst_v.at[j]], add=True)
        return 0

    lax.fori_loop(0, CPW, chunk, 0)

    pltpu.sync_copy(nv_hbm.at[wid], nv_v)
    for j in range(NVC):
        pltpu.async_copy(tab_hbm.at[nv_v.at[j]], emb_v, gsem).wait()
        pltpu.sync_copy(emb_v, emb_out.at[pl.ds(wid * (NVC * CH) + j * CH, CH)])

    plsc.subcore_barrier()
    pltpu.sync_copy(pool_sh.at[pl.ds(sid * OPT, OPT)],
                    pool_out.at[cid, pl.ds(sid * OPT, OPT)])


_pro_call = functools.partial(
    pl.kernel,
    out_type=(jax.ShapeDtypeStruct((NC, NO, LAT), _f32),
              jax.ShapeDtypeStruct((NVP, LAT), _f32)),
    mesh=_mesh,
    scratch_types=[
        pltpu.VMEM((CPW, CH), jnp.int32),
        pltpu.VMEM((CH, LAT), _f32),
        pltpu.VMEM((DE, CH), _f32),
        pltpu.VMEM((NVC, CH), jnp.int32),
        pltpu.VMEM((CH, LAT), _f32),
        pltpu.VMEM_SHARED((NP, LAT), _f32),
        pltpu.SemaphoreType.DMA,
    ],
)(_pro_body)


def _bn_tc(x, g, b):
    m = jnp.mean(x, axis=0, keepdims=True)
    v = jnp.mean((x - m) ** 2, axis=0, keepdims=True)
    return (x - m) / jnp.sqrt(v + EPS) * g + b


def _h0_body(nf, emb, pp, wn, wnb, w0e, g0, b0, h_out, pool_out):
    pool = pp[0, 0:N, :] + pp[1, 0:N, :]
    x = jnp.dot(nf[...], wn[...], preferred_element_type=_f32)
    x = x + wnb[...] + emb[...] + jnp.dot(pool, w0e[...],
                                          preferred_element_type=_f32)
    x = jnp.maximum(x, 0.0)
    h_out[...] = _bn_tc(x, g0[...], b0[...])
    pool_out[...] = pool


def _upd_core(h_ref, a_ref, pool, cw, cb, wle, l2w, l2b, hg, hb, mg, mb):
    agg = a_ref[0, 0:N, :] + a_ref[1, 0:N, :]
    nl = jnp.dot(agg, cw[...], preferred_element_type=_f32) + cb[...]
    ep = jnp.dot(pool[...], wle[...], preferred_element_type=_f32)
    merged = jnp.maximum(nl + ep, 0.0)
    merged = _bn_tc(merged, hg[...], hb[...])
    x = jnp.dot(merged, l2w[...], preferred_element_type=_f32) + l2b[...] + h_ref[...]
    x = jnp.maximum(x, 0.0)
    return _bn_tc(x, mg[...], mb[...])


def _upd_body(h_ref, a_ref, pool, cw, cb, wle, l2w, l2b, hg, hb, mg, mb, out):
    out[...] = _upd_core(h_ref, a_ref, pool, cw, cb, wle, l2w, l2b, hg, hb,
                         mg, mb)


def _upd_ro_body(h_ref, a_ref, pool, cw, cb, wle, l2w, l2b, hg, hb, mg, mb,
                 gidx, row, rob, out):
    x = _upd_core(h_ref, a_ref, pool, cw, cb, wle, l2w, l2b, hg, hb, mg, mb)
    g = gidx[...]
    rows = []
    for gg in range(NG):
        m = g == gg
        rows.append(jnp.max(jnp.where(m, x, -jnp.inf), axis=0, keepdims=True))
    pooled = jnp.concatenate(rows, axis=0)
    out[...] = jnp.maximum(
        jnp.dot(pooled, row[...], preferred_element_type=_f32) + rob[...], 0.0)


def kernel(node_feat, edge_feat, edge_index, node_val_idx, g_idx,
           w_n2l_W, w_n2l_b, node_val_emb, we_W, we_b,
           conv_W, conv_b, l2_W, l2_b,
           msg_g, msg_b, hid_g, hid_b, ro_W, ro_b):
    src = edge_index[0].astype(jnp.int32)
    dst = edge_index[1].astype(jnp.int32)
    pad = EP - E
    ar = jnp.arange(pad, dtype=jnp.int32)
    src3 = jnp.concatenate([src, (ar * 131) % N]).reshape(NW, CPW, CH)
    dst3 = jnp.concatenate([dst, N + (ar % PADR)]).reshape(NW, CPW, CH)

    ef16 = jnp.zeros((EP, DE), _f32).at[:E].set(edge_feat)
    ef4 = ef16.reshape(NW, CPW, DE, CH)

    nvp = jnp.concatenate([
        node_val_idx.astype(jnp.int32),
        (jnp.arange(NVP - N, dtype=jnp.int32) * 7) % NVAL,
    ]).reshape(NW, NVC, CH)

    pool_p, emb_p = _pro_call(ef4, dst3, nvp, node_val_emb)
    emb = emb_p[:N]

    wext = jnp.zeros((MAXLV + 1, LAT, LAT), _f32)
    wext = wext.at[:, :DE, :].set(we_W).at[:, DE, :].set(we_b)

    r = lambda a: a.reshape(1, -1)

    h0_call = pl.pallas_call(
        _h0_body,
        out_shape=(jax.ShapeDtypeStruct((N, LAT), _f32),
                   jax.ShapeDtypeStruct((N, LAT), _f32)),
    )
    h, pool_sum = h0_call(node_feat, emb, pool_p, w_n2l_W, r(w_n2l_b),
                          wext[0], r(msg_g[0]), r(msg_b[0]))

    out = None
    for lv in range(MAXLV):
        a = _agg_call(h, src3, dst3)
        args = (h, a, pool_sum, conv_W[lv], r(conv_b[lv]), wext[lv + 1],
                l2_W[lv], r(l2_b[lv]), r(hid_g[lv]), r(hid_b[lv]),
                r(msg_g[lv + 1]), r(msg_b[lv + 1]))
        if lv < MAXLV - 1:
            h = pl.pallas_call(
                _upd_body,
                out_shape=jax.ShapeDtypeStruct((N, LAT), _f32),
            )(*args)
        else:
            out = pl.pallas_call(
                _upd_ro_body,
                out_shape=jax.ShapeDtypeStruct((NG, ODIM), _f32),
            )(*args, g_idx.reshape(N, 1).astype(jnp.int32), ro_W, r(ro_b))
    return out
